# baseline (speedup 1.0000x reference)
import functools

import jax
import jax.numpy as jnp
from jax import lax
from jax.experimental import pallas as pl
from jax.experimental.pallas import tpu as pltpu
from jax.experimental.pallas import tpu_sc as plsc

NUM_CORES = 2
NUM_SUBCORES = 16
LANES = 16
NUM_WORKERS = NUM_CORES * NUM_SUBCORES

BATCH = 16384
EMBED_DIM = 32
ROWS_PER_WORKER = BATCH // NUM_WORKERS
GATHER_CHUNK = 128
NUM_CHUNKS = ROWS_PER_WORKER // GATHER_CHUNK
OUT_GROUPS = ROWS_PER_WORKER // LANES


def _cf_body(user_ids_hbm, item_ids_hbm, user_table_hbm, item_table_hbm,
             out_hbm, uidx_v, iidx_v, urows_v, irows_v, out_v, sem):
  wid = lax.axis_index("s") * NUM_CORES + lax.axis_index("c")
  base = wid * ROWS_PER_WORKER

  pltpu.sync_copy(user_ids_hbm.at[wid], uidx_v)
  pltpu.sync_copy(item_ids_hbm.at[wid], iidx_v)

  copies = []
  for j in range(NUM_CHUNKS):
    copies.append(pltpu.async_copy(
        user_table_hbm.at[uidx_v.at[j]],
        urows_v.at[pl.ds(j * GATHER_CHUNK, GATHER_CHUNK)], sem))
    copies.append(pltpu.async_copy(
        item_table_hbm.at[iidx_v.at[j]],
        irows_v.at[pl.ds(j * GATHER_CHUNK, GATHER_CHUNK)], sem))
  for c in copies:
    c.wait()

  lane = lax.iota(jnp.int32, LANES)

  def group_body(g, carry):
    rows = g * LANES + lane
    acc = jnp.zeros((LANES,), jnp.float32)
    for d in range(EMBED_DIM):
      col = jnp.full((LANES,), d, jnp.int32)
      u = plsc.load_gather(urows_v, [rows, col])
      v = plsc.load_gather(irows_v, [rows, col])
      acc = acc + u * v
    out_v[pl.ds(g * LANES, LANES)] = acc
    return carry

  lax.fori_loop(0, OUT_GROUPS, group_body, 0)

  pltpu.sync_copy(out_v, out_hbm.at[pl.ds(base, ROWS_PER_WORKER)])


@jax.jit
def _cf_scores(user_ids, item_ids, user_table, item_table):
  mesh = plsc.VectorSubcoreMesh(
      core_axis_name="c", subcore_axis_name="s",
      num_cores=NUM_CORES, num_subcores=NUM_SUBCORES)
  return pl.kernel(
      _cf_body,
      out_type=jax.ShapeDtypeStruct((BATCH,), jnp.float32),
      mesh=mesh,
      compiler_params=pltpu.CompilerParams(
          needs_layout_passes=False, use_tc_tiling_on_sc=False),
      scratch_types=[
          pltpu.VMEM((NUM_CHUNKS, GATHER_CHUNK), jnp.int32),
          pltpu.VMEM((NUM_CHUNKS, GATHER_CHUNK), jnp.int32),
          pltpu.VMEM((ROWS_PER_WORKER, EMBED_DIM), jnp.float32),
          pltpu.VMEM((ROWS_PER_WORKER, EMBED_DIM), jnp.float32),
          pltpu.VMEM((ROWS_PER_WORKER,), jnp.float32),
          pltpu.SemaphoreType.DMA,
      ],
  )(user_ids, item_ids, user_table, item_table)


def kernel(user_ids, item_ids, user_table, item_table):
  uids = user_ids.astype(jnp.int32).reshape(NUM_WORKERS, NUM_CHUNKS,
                                            GATHER_CHUNK)
  iids = item_ids.astype(jnp.int32).reshape(NUM_WORKERS, NUM_CHUNKS,
                                            GATHER_CHUNK)
  return _cf_scores(uids, iids, user_table, item_table)


# speedup vs baseline: 3.3504x; 3.3504x over previous
import jax
import jax.numpy as jnp
from jax import lax
from jax.experimental import pallas as pl
from jax.experimental.pallas import tpu as pltpu
from jax.experimental.pallas import tpu_sc as plsc

NC, NS, L = 2, 16, 16
NW = NC * NS
B = 16384
D = 32
DG, DS = 4, 8
BPW = B // NW
WV = 16
NWAVE = BPW // WV


def _body(uids_hbm, iids_hbm, ut_hbm, it_hbm, out_hbm,
          wuids_v, wiids_v, blk_v, uval_v, ow_v, sem):
  wid = lax.axis_index("s") * NC + lax.axis_index("c")
  base = wid * BPW

  lane = lax.iota(jnp.int32, L)

  def wave(wv, carry):
    pltpu.sync_copy(uids_hbm.at[wid, pl.ds(wv * L, L)], wuids_v)
    pltpu.sync_copy(iids_hbm.at[wid, pl.ds(wv * L, L)], wiids_v)
    uvec = wuids_v[...]
    ivec = wiids_v[...]

    def fire(vec):
      def fire_one(j, c):
        r = jnp.sum(jnp.where(lane == j, vec, 0))
        a = pl.multiple_of((r >> 7) << 7, 128)
        for dg in range(DG):
          pltpu.async_copy(ut_hbm.at[dg, :, pl.ds(a, 128)],
                           blk_v.at[j * DG + dg], sem)
        return c
      return fire_one

    def fire_one_item(j, c):
      r = jnp.sum(jnp.where(lane == j, ivec, 0))
      a = pl.multiple_of((r >> 7) << 7, 128)
      for dg in range(DG):
        pltpu.async_copy(it_hbm.at[dg, :, pl.ds(a, 128)],
                         blk_v.at[j * DG + dg], sem)
      return c

    def drain_one(t, c):
      pltpu.make_async_copy(ut_hbm.at[0, :, pl.ds(0, 128)],
                            blk_v.at[0], sem).wait()
      return c

    lax.fori_loop(0, WV, fire(uvec), 0)
    lax.fori_loop(0, WV * DG, drain_one, 0)
    ucol = jnp.bitwise_and(uvec, 127)
    rows = lane * DG
    for d in range(D):
      dg, s = d // DS, d % DS
      sv = jnp.full((L,), s, jnp.int32)
      uval_v[pl.ds(d * L, L)] = plsc.load_gather(blk_v,
                                                 [rows + dg, sv, ucol])

    lax.fori_loop(0, WV, fire_one_item, 0)
    lax.fori_loop(0, WV * DG, drain_one, 0)
    icol = jnp.bitwise_and(ivec, 127)
    acc = jnp.zeros((L,), jnp.float32)
    for d in range(D):
      dg, s = d // DS, d % DS
      sv = jnp.full((L,), s, jnp.int32)
      iv = plsc.load_gather(blk_v, [rows + dg, sv, icol])
      acc = acc + uval_v[pl.ds(d * L, L)] * iv

    ow_v[...] = acc
    pltpu.sync_copy(ow_v, out_hbm.at[pl.ds(base + wv * L, L)])
    return carry

  lax.fori_loop(0, NWAVE, wave, 0)


@jax.jit
def _scores(uids, iids, ut3, it3):
  mesh = plsc.VectorSubcoreMesh(
      core_axis_name="c", subcore_axis_name="s", num_cores=NC, num_subcores=NS)
  return pl.kernel(
      _body,
      out_type=jax.ShapeDtypeStruct((B,), jnp.float32),
      mesh=mesh,
      compiler_params=pltpu.CompilerParams(needs_layout_passes=False),
      scratch_types=[
          pltpu.VMEM((L,), jnp.int32),
          pltpu.VMEM((L,), jnp.int32),
          pltpu.VMEM((WV * DG, DS, 128), jnp.float32),
          pltpu.VMEM((D * L,), jnp.float32),
          pltpu.VMEM((L,), jnp.float32),
          pltpu.SemaphoreType.DMA,
      ],
  )(uids, iids, ut3, it3)


def kernel(user_ids, item_ids, user_table, item_table):
  uids = user_ids.astype(jnp.int32).reshape(NW, BPW)
  iids = item_ids.astype(jnp.int32).reshape(NW, BPW)
  ut3 = user_table.T.reshape(DG, DS, 1000000)
  it3 = item_table.T.reshape(DG, DS, 1000000)
  return _scores(uids, iids, ut3, it3)


# speedup vs baseline: 7.1299x; 2.1281x over previous
import jax
import jax.numpy as jnp
from jax import lax
from jax.experimental import pallas as pl
from jax.experimental.pallas import tpu as pltpu
from jax.experimental.pallas import tpu_sc as plsc

NC, NS, L = 2, 16, 16
NW = NC * NS
B = 16384
D = 32
DS = 8
BPW = B // NW
WV = 64
NWAVE = BPW // WV
NBODY = WV * 4
NGRP = WV // L


def _body(uids_hbm, iids_hbm, ut_hbm, it_hbm, out_hbm,
          uids_v, iids_v, ublk_v, iblk_v, out_v, sem):
  wid = lax.axis_index("s") * NC + lax.axis_index("c")
  base = wid * BPW

  pltpu.sync_copy(uids_hbm.at[wid], uids_v)
  pltpu.sync_copy(iids_hbm.at[wid], iids_v)

  lane = lax.iota(jnp.int32, L)

  for w in range(NWAVE):
    def fire_body(t, carry, w=w):
      idl = t // 4
      q = t % 4
      gidx = w * WV + idl
      gbase = (gidx // L) * L
      k = gidx - gbase
      uvec = uids_v[pl.ds(gbase, L)]
      ivec = iids_v[pl.ds(gbase, L)]
      ru = jnp.sum(jnp.where(lane == k, uvec, 0))
      ri = jnp.sum(jnp.where(lane == k, ivec, 0))
      au = pl.multiple_of((ru >> 4) << 4, 16)
      ai = pl.multiple_of((ri >> 4) << 4, 16)
      for dd in range(DS):
        d = q * DS + dd
        row = idl * 4 + d // 8
        col = (d % 8) * 16
        pltpu.async_copy(ut_hbm.at[d].at[pl.ds(au, 16)],
                         ublk_v.at[row, pl.ds(col, 16)], sem)
        pltpu.async_copy(it_hbm.at[d].at[pl.ds(ai, 16)],
                         iblk_v.at[row, pl.ds(col, 16)], sem)
      return carry

    lax.fori_loop(0, NBODY, fire_body, 0)

    def drain_one(t, c):
      pltpu.make_async_copy(ut_hbm.at[0].at[pl.ds(0, 128)],
                            ublk_v.at[0], sem).wait()
      return c

    lax.fori_loop(0, 2 * WV * 4, drain_one, 0)

    def dot_group(gi, carry, w=w):
      idl = gi * L + lane
      gstart = w * WV + gi * L
      cu = jnp.bitwise_and(uids_v[pl.ds(gstart, L)], 15)
      ci = jnp.bitwise_and(iids_v[pl.ds(gstart, L)], 15)
      acc = jnp.zeros((L,), jnp.float32)
      for d in range(D):
        row = idl * 4 + d // 8
        col = (d % 8) * 16
        u = plsc.load_gather(ublk_v, [row, col + cu])
        v = plsc.load_gather(iblk_v, [row, col + ci])
        acc = acc + u * v
      out_v[pl.ds(gstart, L)] = acc
      return carry

    lax.fori_loop(0, NGRP, dot_group, 0)

  pltpu.sync_copy(out_v, out_hbm.at[pl.ds(base, BPW)])


@jax.jit
def _scores(uids, iids, ut_t, it_t):
  mesh = plsc.VectorSubcoreMesh(
      core_axis_name="c", subcore_axis_name="s", num_cores=NC, num_subcores=NS)
  return pl.kernel(
      _body,
      out_type=jax.ShapeDtypeStruct((B,), jnp.float32),
      mesh=mesh,
      compiler_params=pltpu.CompilerParams(needs_layout_passes=False),
      scratch_types=[
          pltpu.VMEM((BPW,), jnp.int32),
          pltpu.VMEM((BPW,), jnp.int32),
          pltpu.VMEM((WV * 4, 128), jnp.float32),
          pltpu.VMEM((WV * 4, 128), jnp.float32),
          pltpu.VMEM((BPW,), jnp.float32),
          pltpu.SemaphoreType.DMA,
      ],
  )(uids, iids, ut_t, it_t)


def kernel(user_ids, item_ids, user_table, item_table):
  uids = user_ids.astype(jnp.int32).reshape(NW, BPW)
  iids = item_ids.astype(jnp.int32).reshape(NW, BPW)
  return _scores(uids, iids, user_table.T, item_table.T)


# speedup vs baseline: 8.7903x; 1.2329x over previous
import jax
import jax.numpy as jnp
from jax import lax
from jax.experimental import pallas as pl
from jax.experimental.pallas import tpu as pltpu
from jax.experimental.pallas import tpu_sc as plsc

NC, NS, L = 2, 16, 16
NW = NC * NS
B = 16384
D = 32
DS = 8
BPW = B // NW
WV = 64
NWAVE = BPW // WV
NBODY = WV * 4
NGRP = WV // L


def _body(uids_hbm, iids_hbm, ut_hbm, it_hbm, out_hbm,
          uids_v, iids_v, ublk_v, iblk_v, out_v, sem):
  wid = lax.axis_index("s") * NC + lax.axis_index("c")
  base = wid * BPW

  pltpu.sync_copy(uids_hbm.at[wid], uids_v)
  pltpu.sync_copy(iids_hbm.at[wid], iids_v)

  lane = lax.iota(jnp.int32, L)

  for w in range(NWAVE):
    def fire_body(j, carry, w=w):
      gidx = w * WV + j
      gbase = (gidx // L) * L
      k = gidx - gbase
      uvec = uids_v[pl.ds(gbase, L)]
      ivec = iids_v[pl.ds(gbase, L)]
      ru = jnp.sum(jnp.where(lane == k, uvec, 0))
      ri = jnp.sum(jnp.where(lane == k, ivec, 0))
      au = pl.multiple_of((ru >> 4) << 4, 16)
      ai = pl.multiple_of((ri >> 4) << 4, 16)
      for d in range(D):
        row = j * 4 + d // 8
        col = (d % 8) * 16
        pltpu.async_copy(ut_hbm.at[d].at[pl.ds(au, 16)],
                         ublk_v.at[row, pl.ds(col, 16)], sem)
        pltpu.async_copy(it_hbm.at[d].at[pl.ds(ai, 16)],
                         iblk_v.at[row, pl.ds(col, 16)], sem)
      return carry

    lax.fori_loop(0, WV, fire_body, 0)

    def drain_one(t, c):
      pltpu.make_async_copy(ut_hbm.at[0].at[pl.ds(0, 128)],
                            ublk_v.at[0], sem).wait()
      return c

    lax.fori_loop(0, 2 * WV * 4, drain_one, 0)

    def dot_group(gi, carry, w=w):
      idl = gi * L + lane
      gstart = w * WV + gi * L
      cu = jnp.bitwise_and(uids_v[pl.ds(gstart, L)], 15)
      ci = jnp.bitwise_and(iids_v[pl.ds(gstart, L)], 15)
      acc = jnp.zeros((L,), jnp.float32)
      for d in range(D):
        row = idl * 4 + d // 8
        col = (d % 8) * 16
        u = plsc.load_gather(ublk_v, [row, col + cu])
        v = plsc.load_gather(iblk_v, [row, col + ci])
        acc = acc + u * v
      out_v[pl.ds(gstart, L)] = acc
      return carry

    lax.fori_loop(0, NGRP, dot_group, 0)

  pltpu.sync_copy(out_v, out_hbm.at[pl.ds(base, BPW)])


@jax.jit
def _scores(uids, iids, ut_t, it_t):
  mesh = plsc.VectorSubcoreMesh(
      core_axis_name="c", subcore_axis_name="s", num_cores=NC, num_subcores=NS)
  return pl.kernel(
      _body,
      out_type=jax.ShapeDtypeStruct((B,), jnp.float32),
      mesh=mesh,
      compiler_params=pltpu.CompilerParams(needs_layout_passes=False),
      scratch_types=[
          pltpu.VMEM((BPW,), jnp.int32),
          pltpu.VMEM((BPW,), jnp.int32),
          pltpu.VMEM((WV * 4, 128), jnp.float32),
          pltpu.VMEM((WV * 4, 128), jnp.float32),
          pltpu.VMEM((BPW,), jnp.float32),
          pltpu.SemaphoreType.DMA,
      ],
  )(uids, iids, ut_t, it_t)


def kernel(user_ids, item_ids, user_table, item_table):
  uids = user_ids.astype(jnp.int32).reshape(NW, BPW)
  iids = item_ids.astype(jnp.int32).reshape(NW, BPW)
  return _scores(uids, iids, user_table.T, item_table.T)


# speedup vs baseline: 9.6189x; 1.0943x over previous
import jax
import jax.numpy as jnp
from jax import lax
from jax.experimental import pallas as pl
from jax.experimental.pallas import tpu as pltpu
from jax.experimental.pallas import tpu_sc as plsc

NC, NS, L = 2, 16, 16
NW = NC * NS
B = 16384
D = 32
DS = 8
BPW = B // NW
WV = 64
NWAVE = BPW // WV
NBODY = WV * 4
NGRP = WV // L


def _body(uids_hbm, iids_hbm, ut_hbm, it_hbm, out_hbm,
          uids_v, iids_v, ublk_v, iblk_v, out_v, sem):
  wid = lax.axis_index("s") * NC + lax.axis_index("c")
  base = wid * BPW

  pltpu.sync_copy(uids_hbm.at[wid], uids_v)
  pltpu.sync_copy(iids_hbm.at[wid], iids_v)

  lane = lax.iota(jnp.int32, L)

  for w in range(NWAVE):
    def fire_body(j, carry, w=w):
      gidx = w * WV + j
      gbase = (gidx // L) * L
      k = gidx - gbase
      uvec = uids_v[pl.ds(gbase, L)]
      ivec = iids_v[pl.ds(gbase, L)]
      ru = jnp.sum(jnp.where(lane == k, uvec, 0))
      ri = jnp.sum(jnp.where(lane == k, ivec, 0))
      au = pl.multiple_of((ru >> 3) << 3, 8)
      ai = pl.multiple_of((ri >> 3) << 3, 8)
      for d in range(D):
        row = j * 2 + d // 16
        col = (d % 16) * 8
        pltpu.async_copy(ut_hbm.at[d].at[pl.ds(au, 8)],
                         ublk_v.at[row, pl.ds(col, 8)], sem)
        pltpu.async_copy(it_hbm.at[d].at[pl.ds(ai, 8)],
                         iblk_v.at[row, pl.ds(col, 8)], sem)
      return carry

    lax.fori_loop(0, WV, fire_body, 0)

    def drain_one(t, c):
      pltpu.make_async_copy(ut_hbm.at[0].at[pl.ds(0, 128)],
                            ublk_v.at[0], sem).wait()
      return c

    lax.fori_loop(0, WV * 4, drain_one, 0)

    def dot_group(gi, carry, w=w):
      idl = gi * L + lane
      gstart = w * WV + gi * L
      cu = jnp.bitwise_and(uids_v[pl.ds(gstart, L)], 7)
      ci = jnp.bitwise_and(iids_v[pl.ds(gstart, L)], 7)
      acc = jnp.zeros((L,), jnp.float32)
      for d in range(D):
        row = idl * 2 + d // 16
        col = (d % 16) * 8
        u = plsc.load_gather(ublk_v, [row, col + cu])
        v = plsc.load_gather(iblk_v, [row, col + ci])
        acc = acc + u * v
      out_v[pl.ds(gstart, L)] = acc
      return carry

    lax.fori_loop(0, NGRP, dot_group, 0)

  pltpu.sync_copy(out_v, out_hbm.at[pl.ds(base, BPW)])


@jax.jit
def _scores(uids, iids, ut_t, it_t):
  mesh = plsc.VectorSubcoreMesh(
      core_axis_name="c", subcore_axis_name="s", num_cores=NC, num_subcores=NS)
  return pl.kernel(
      _body,
      out_type=jax.ShapeDtypeStruct((B,), jnp.float32),
      mesh=mesh,
      compiler_params=pltpu.CompilerParams(needs_layout_passes=False),
      scratch_types=[
          pltpu.VMEM((BPW,), jnp.int32),
          pltpu.VMEM((BPW,), jnp.int32),
          pltpu.VMEM((WV * 2, 128), jnp.float32),
          pltpu.VMEM((WV * 2, 128), jnp.float32),
          pltpu.VMEM((BPW,), jnp.float32),
          pltpu.SemaphoreType.DMA,
      ],
  )(uids, iids, ut_t, it_t)


def kernel(user_ids, item_ids, user_table, item_table):
  uids = user_ids.astype(jnp.int32).reshape(NW, BPW)
  iids = item_ids.astype(jnp.int32).reshape(NW, BPW)
  return _scores(uids, iids, user_table.T, item_table.T)


# speedup vs baseline: 9.6372x; 1.0019x over previous
import jax
import jax.numpy as jnp
from jax import lax
from jax.experimental import pallas as pl
from jax.experimental.pallas import tpu as pltpu
from jax.experimental.pallas import tpu_sc as plsc

NC, NS, L = 2, 16, 16
NW = NC * NS
B = 16384
D = 32
DS = 8
BPW = B // NW
WV = 64
NWAVE = BPW // WV
NBODY = WV * 4
NGRP = WV // L


def _body(uids_hbm, iids_hbm, ut_hbm, it_hbm, out_hbm,
          uids_v, iids_v, ublk0_v, iblk0_v, ublk1_v, iblk1_v, out_v,
          sem0, sem1):
  wid = lax.axis_index("s") * NC + lax.axis_index("c")
  base = wid * BPW

  pltpu.sync_copy(uids_hbm.at[wid], uids_v)
  pltpu.sync_copy(iids_hbm.at[wid], iids_v)

  lane = lax.iota(jnp.int32, L)
  bufs = ((ublk0_v, iblk0_v, sem0), (ublk1_v, iblk1_v, sem1))

  def do_fire(w, ublk_v, iblk_v, sem):
    def fire_body(j, carry):
      gidx = w * WV + j
      gbase = (gidx // L) * L
      k = gidx - gbase
      uvec = uids_v[pl.ds(gbase, L)]
      ivec = iids_v[pl.ds(gbase, L)]
      ru = jnp.sum(jnp.where(lane == k, uvec, 0))
      ri = jnp.sum(jnp.where(lane == k, ivec, 0))
      au = pl.multiple_of((ru >> 3) << 3, 8)
      ai = pl.multiple_of((ri >> 3) << 3, 8)
      for d in range(D):
        row = j * 2 + d // 16
        col = (d % 16) * 8
        pltpu.async_copy(ut_hbm.at[d].at[pl.ds(au, 8)],
                         ublk_v.at[row, pl.ds(col, 8)], sem)
        pltpu.async_copy(it_hbm.at[d].at[pl.ds(ai, 8)],
                         iblk_v.at[row, pl.ds(col, 8)], sem)
      return carry

    lax.fori_loop(0, WV, fire_body, 0)

  def do_drain(ublk_v, sem):
    def drain_one(t, c):
      pltpu.make_async_copy(ut_hbm.at[0].at[pl.ds(0, 128)],
                            ublk_v.at[0], sem).wait()
      return c

    lax.fori_loop(0, WV * 4, drain_one, 0)

  def do_compute(w, ublk_v, iblk_v):
    def dot_group(gi, carry):
      idl = gi * L + lane
      gstart = w * WV + gi * L
      cu = jnp.bitwise_and(uids_v[pl.ds(gstart, L)], 7)
      ci = jnp.bitwise_and(iids_v[pl.ds(gstart, L)], 7)
      acc = jnp.zeros((L,), jnp.float32)
      for d in range(D):
        row = idl * 2 + d // 16
        col = (d % 16) * 8
        u = plsc.load_gather(ublk_v, [row, col + cu])
        v = plsc.load_gather(iblk_v, [row, col + ci])
        acc = acc + u * v
      out_v[pl.ds(gstart, L)] = acc
      return carry

    lax.fori_loop(0, NGRP, dot_group, 0)

  do_fire(0, *bufs[0])
  for w in range(NWAVE):
    if w + 1 < NWAVE:
      do_fire(w + 1, *bufs[(w + 1) & 1])
    do_drain(bufs[w & 1][0], bufs[w & 1][2])
    do_compute(w, bufs[w & 1][0], bufs[w & 1][1])

  pltpu.sync_copy(out_v, out_hbm.at[pl.ds(base, BPW)])


@jax.jit
def _scores(uids, iids, ut_t, it_t):
  mesh = plsc.VectorSubcoreMesh(
      core_axis_name="c", subcore_axis_name="s", num_cores=NC, num_subcores=NS)
  return pl.kernel(
      _body,
      out_type=jax.ShapeDtypeStruct((B,), jnp.float32),
      mesh=mesh,
      compiler_params=pltpu.CompilerParams(needs_layout_passes=False),
      scratch_types=[
          pltpu.VMEM((BPW,), jnp.int32),
          pltpu.VMEM((BPW,), jnp.int32),
          pltpu.VMEM((WV * 2, 128), jnp.float32),
          pltpu.VMEM((WV * 2, 128), jnp.float32),
          pltpu.VMEM((WV * 2, 128), jnp.float32),
          pltpu.VMEM((WV * 2, 128), jnp.float32),
          pltpu.VMEM((BPW,), jnp.float32),
          pltpu.SemaphoreType.DMA,
          pltpu.SemaphoreType.DMA,
      ],
  )(uids, iids, ut_t, it_t)


def kernel(user_ids, item_ids, user_table, item_table):
  uids = user_ids.astype(jnp.int32).reshape(NW, BPW)
  iids = item_ids.astype(jnp.int32).reshape(NW, BPW)
  return _scores(uids, iids, user_table.T, item_table.T)
